# baseline (speedup 1.0000x reference)
import functools

import jax
import jax.numpy as jnp
from jax import lax
from jax.experimental import pallas as pl
from jax.experimental.pallas import tpu as pltpu
from jax.experimental.pallas import tpu_sc as plsc

BATCH = 16384
EMB_K = 16

_NC = 2
_NS = 16
_L = 16
_NW = _NC * _NS
_BPW = BATCH // _NW
_CHUNK = 128
_NCHUNK = _BPW // _CHUNK
_NBLK = _BPW // _L


def _ncf_body(uidx_hbm, vidx_hbm, w_hbm, h_hbm, w1_hbm, b1_hbm, w2_hbm,
              out_hbm,
              uidx_v, vidx_v, urows_v, vrows_v,
              w1_v, b1_v, w2_v, out_v,
              sem_u, sem_v):
    wid = lax.axis_index("s") * _NC + lax.axis_index("c")
    crow = wid * _NCHUNK
    pltpu.sync_copy(uidx_hbm.at[pl.ds(crow, _NCHUNK)], uidx_v)
    pltpu.sync_copy(vidx_hbm.at[pl.ds(crow, _NCHUNK)], vidx_v)

    copies = []
    for c in range(_NCHUNK):
        copies.append(pltpu.async_copy(
            w_hbm.at[uidx_v.at[c]],
            urows_v.at[pl.ds(c * _CHUNK, _CHUNK)], sem_u))
        copies.append(pltpu.async_copy(
            h_hbm.at[vidx_v.at[c]],
            vrows_v.at[pl.ds(c * _CHUNK, _CHUNK)], sem_v))
    pltpu.sync_copy(w1_hbm, w1_v)
    pltpu.sync_copy(b1_hbm, b1_v)
    pltpu.sync_copy(w2_hbm, w2_v)
    for cp in copies:
        cp.wait()

    lane = lax.iota(jnp.int32, _L)
    b1_vec = b1_v[...]
    w2_vec = w2_v[...]

    def block(blk, carry):
        row_ids = blk * _L + lane
        ucols = [plsc.load_gather(urows_v,
                                  [row_ids, jnp.full((_L,), k, jnp.int32)])
                 for k in range(EMB_K)]
        vcols = [plsc.load_gather(vrows_v,
                                  [row_ids, jnp.full((_L,), k, jnp.int32)])
                 for k in range(EMB_K)]
        acc = jnp.zeros((_L,), jnp.float32)
        for j in range(EMB_K):
            w1u = w1_v[j, pl.ds(0, EMB_K)]
            w1v = w1_v[j, pl.ds(EMB_K, EMB_K)]
            h = jnp.full((_L,), b1_vec[j], jnp.float32)
            for k in range(EMB_K):
                h = h + ucols[k] * w1u[k]
            for k in range(EMB_K):
                h = h + vcols[k] * w1v[k]
            h = jnp.maximum(h, 0.0)
            acc = acc + h * w2_vec[j]
        out_v[pl.ds(blk * _L, _L)] = acc
        return carry

    lax.fori_loop(0, _NBLK, block, 0)

    pltpu.sync_copy(out_v, out_hbm.at[pl.ds(wid * _BPW, _BPW)])


@functools.cache
def _make_ncf_sc():
  return functools.partial(
    pl.kernel,
    out_type=jax.ShapeDtypeStruct((BATCH,), jnp.float32),
    mesh=plsc.VectorSubcoreMesh(core_axis_name="c", subcore_axis_name="s",
                                num_cores=_NC),
    compiler_params=pltpu.CompilerParams(needs_layout_passes=False,
                                         use_tc_tiling_on_sc=False),
    scratch_types=[
        pltpu.VMEM((_NCHUNK, _CHUNK), jnp.int32),
        pltpu.VMEM((_NCHUNK, _CHUNK), jnp.int32),
        pltpu.VMEM((_BPW, EMB_K), jnp.float32),
        pltpu.VMEM((_BPW, EMB_K), jnp.float32),
        pltpu.VMEM((EMB_K, 2 * EMB_K), jnp.float32),
        pltpu.VMEM((EMB_K,), jnp.float32),
        pltpu.VMEM((EMB_K,), jnp.float32),
        pltpu.VMEM((_BPW,), jnp.float32),
        pltpu.SemaphoreType.DMA,
        pltpu.SemaphoreType.DMA,
    ],
  )(_ncf_body)


def kernel(x, W_table, H_table, W1, b1, W2):
    u_idx = x[:, 0].reshape(BATCH // _CHUNK, _CHUNK)
    v_idx = x[:, 1].reshape(BATCH // _CHUNK, _CHUNK)
    out = _make_ncf_sc()(u_idx, v_idx, W_table, H_table, W1, b1,
                         W2.reshape(EMB_K))
    return out.reshape(BATCH, 1)
